# Initial kernel scaffold; baseline (speedup 1.0000x reference)
import jax
import jax.numpy as jnp
from jax.experimental import pallas as pl


def kernel(x_nchw, w1k, b1r, w2k, b2r, fc1a_w, fc1_b, fc2a_w, fc2_b, fc3a_w, fc3_b):
    raise NotImplementedError("write your pallas kernel here")


# speedup vs baseline: 2.1197x; 2.1197x over previous
import jax
import jax.numpy as jnp
from jax.experimental import pallas as pl
from jax.experimental.pallas import tpu as pltpu

_BF = jnp.bfloat16
_F32 = jnp.float32


def _conv_tower_kernel(x_ref, w1_ref, b1_ref, w2_ref, b2_ref, f_ref,
                       p0_ref, p1_ref, s2_ref):
    bn = x_ref.shape[0]

    def body(i, carry):
        xi = x_ref[i]
        cols = []
        for kh in range(5):
            for kw in range(5):
                off = kh * 32 + kw
                cols.append(jax.lax.slice(xi, (off, 0), (off + 892, 8)))
        im1 = jnp.concatenate(cols, axis=1)
        a1 = jnp.dot(im1, w1_ref[...], preferred_element_type=_F32)
        p0_ref[0:892, :] = jnp.maximum(a1 + b1_ref[...], 0.0).astype(_BF)
        p0_ref[892:896, :] = jnp.zeros((4, 64), _BF)
        p1_ref[0:863, :] = jnp.maximum(
            jnp.maximum(p0_ref[0:863, :], p0_ref[1:864, :]),
            jnp.maximum(p0_ref[32:895, :], p0_ref[33:896, :]))
        parts = []
        for kh in range(5):
            for kw in range(5):
                off = kh * 64 + 2 * kw
                parts.append(p1_ref[off:off + 596, :])
        im2 = jnp.concatenate(parts, axis=1)
        s2_ref[...] = jnp.dot(im2, w2_ref[...], preferred_element_type=_F32)
        q = jnp.maximum(
            jnp.maximum(s2_ref[0:529, :], s2_ref[2:531, :]),
            jnp.maximum(s2_ref[64:593, :], s2_ref[66:595, :]))
        rows = []
        for hp in range(5):
            for wp in range(5):
                r0 = 128 * hp + 4 * wp
                rows.append(jax.lax.slice(q, (r0, 0), (r0 + 1, 64)))
        feats = jnp.concatenate(rows, axis=0)
        f_ref[i] = jnp.maximum(feats + b2_ref[...], 0.0).astype(_BF)
        return carry

    jax.lax.fori_loop(0, bn, body, 0)


def _conv_tower(x_bf, w1, b1, w2, b2):
    B = x_bf.shape[0]
    bn = 1
    for cand in (16, 8, 4, 2):
        if B % cand == 0:
            bn = cand
            break
    return pl.pallas_call(
        _conv_tower_kernel,
        out_shape=jax.ShapeDtypeStruct((B, 25, 64), _BF),
        grid=(B // bn,),
        in_specs=[
            pl.BlockSpec((bn, 1024, 8), lambda b: (b, 0, 0)),
            pl.BlockSpec((200, 64), lambda b: (0, 0)),
            pl.BlockSpec((1, 64), lambda b: (0, 0)),
            pl.BlockSpec((1600, 64), lambda b: (0, 0)),
            pl.BlockSpec((1, 64), lambda b: (0, 0)),
        ],
        out_specs=pl.BlockSpec((bn, 25, 64), lambda b: (b, 0, 0)),
        scratch_shapes=[
            pltpu.VMEM((896, 64), _BF),
            pltpu.VMEM((864, 64), _BF),
            pltpu.VMEM((596, 64), _F32),
        ],
        compiler_params=pltpu.CompilerParams(
            dimension_semantics=("parallel",)),
    )(x_bf, w1, b1, w2, b2)


def _fc_stack_kernel(x_ref, w1_ref, b1_ref, w2_ref, b2_ref, w3_ref, b3_ref,
                     o_ref):
    h = jnp.dot(x_ref[...], w1_ref[...], preferred_element_type=_F32)
    h = jnp.maximum(h + b1_ref[...], 0.0).astype(_BF)
    h = jnp.dot(h, w2_ref[...], preferred_element_type=_F32)
    h = jnp.maximum(h + b2_ref[...], 0.0).astype(_BF)
    o_ref[...] = jnp.dot(h, w3_ref[...], preferred_element_type=_F32) \
        + b3_ref[...]


def _fc_stack(feats2d, w1, b1, w2, b2, w3, b3):
    B = feats2d.shape[0]
    tm = B
    for cand in (256, 128, 64):
        if B % cand == 0:
            tm = cand
            break
    return pl.pallas_call(
        _fc_stack_kernel,
        out_shape=jax.ShapeDtypeStruct((B, 128), _F32),
        grid=(B // tm,),
        in_specs=[
            pl.BlockSpec((tm, 1600), lambda i: (i, 0)),
            pl.BlockSpec((1600, 384), lambda i: (0, 0)),
            pl.BlockSpec((1, 384), lambda i: (0, 0)),
            pl.BlockSpec((384, 192), lambda i: (0, 0)),
            pl.BlockSpec((1, 192), lambda i: (0, 0)),
            pl.BlockSpec((192, 128), lambda i: (0, 0)),
            pl.BlockSpec((1, 128), lambda i: (0, 0)),
        ],
        out_specs=pl.BlockSpec((tm, 128), lambda i: (i, 0)),
        compiler_params=pltpu.CompilerParams(
            dimension_semantics=("parallel",)),
    )(feats2d, w1, b1, w2, b2, w3, b3)


@jax.jit
def _forward(x_nchw, w1k, b1r, w2k, b2r, fc1a_w, fc1_b, fc2a_w, fc2_b,
             fc3a_w, fc3_b):
    B = x_nchw.shape[0]
    x = jnp.transpose(x_nchw, (0, 2, 3, 1)).reshape(B, 1024, 3)
    x = jnp.pad(x, ((0, 0), (0, 0), (0, 5))).astype(_BF)
    w1 = w1k.reshape(200, 64).astype(_BF)
    w2 = w2k.reshape(1600, 64).astype(_BF)
    feats = _conv_tower(x, w1, b1r, w2, b2r)
    logits = _fc_stack(feats.reshape(B, 1600),
                       fc1a_w.astype(_BF), fc1_b,
                       fc2a_w.astype(_BF), fc2_b,
                       fc3a_w.astype(_BF), fc3_b)
    return logits[:, :10]


def kernel(x_nchw, w1k, b1r, w2k, b2r, fc1a_w, fc1_b, fc2a_w, fc2_b,
           fc3a_w, fc3_b):
    return _forward(x_nchw, w1k, b1r, w2k, b2r, fc1a_w, fc1_b, fc2a_w,
                    fc2_b, fc3a_w, fc3_b)


# speedup vs baseline: 4.4932x; 2.1197x over previous
import jax
import jax.numpy as jnp
from jax.experimental import pallas as pl
from jax.experimental.pallas import tpu as pltpu

_BF = jnp.bfloat16
_F32 = jnp.float32


def _build_im1(srcs):
    cols = []
    for src, sh in srcs:
        col = jax.lax.slice(src, (sh, 0), (512, 8))
        if sh:
            col = jnp.pad(col, ((0, sh), (0, 0)))
        cols.append(col)
    xts = jnp.concatenate(cols, axis=1)
    return jnp.concatenate(
        [jax.lax.slice(xts, (16 * kh, 0), (16 * kh + 448, 40))
         for kh in range(5)],
        axis=1)


def _one_image(xr, w1, b1, w2, b2):
    xe = xr[:, 0:8]
    xo = xr[:, 8:16]
    ime = _build_im1([(xe, 0), (xo, 0), (xe, 1), (xo, 1), (xe, 2)])
    imo = _build_im1([(xo, 0), (xe, 1), (xo, 1), (xe, 2), (xo, 2)])
    pe = jnp.maximum(
        jnp.dot(ime, w1, preferred_element_type=_F32) + b1, 0.0).astype(_BF)
    po = jnp.maximum(
        jnp.dot(imo, w1, preferred_element_type=_F32) + b1, 0.0).astype(_BF)
    p1 = jnp.maximum(
        jnp.maximum(pe[0:430, :], po[0:430, :]),
        jnp.maximum(pe[16:446, :], po[16:446, :]))
    im2 = jnp.concatenate(
        [jax.lax.slice(p1, (32 * kh + kw, 0), (32 * kh + kw + 298, 64))
         for kh in range(5) for kw in range(5)],
        axis=1)
    c2 = jnp.dot(im2, w2, preferred_element_type=_F32)
    c3 = c2.reshape(149, 2, 64)
    se = c3[:, 0, :]
    so = c3[:, 1, :]
    q = jnp.maximum(
        jnp.maximum(se[0:133, :], so[0:133, :]),
        jnp.maximum(se[16:149, :], so[16:149, :]))
    rows = [jax.lax.slice(q, (32 * hp + wp, 0), (32 * hp + wp + 1, 64))
            for hp in range(5) for wp in range(5)]
    feats = jnp.concatenate(rows, axis=0)
    return jnp.maximum(feats + b2, 0.0).astype(_BF)


def _conv_tower_kernel(x_ref, w1_ref, b1_ref, w2_ref, b2_ref, f_ref):
    bn = x_ref.shape[0]
    w1 = w1_ref[...]
    b1 = b1_ref[...]
    w2 = w2_ref[...]
    b2 = b2_ref[...]

    def body(i, carry):
        f_ref[i] = _one_image(x_ref[i], w1, b1, w2, b2)
        return carry

    jax.lax.fori_loop(0, bn, body, 0)


def _conv_tower(x_bf, w1, b1, w2, b2):
    B = x_bf.shape[0]
    bn = 1
    for cand in (16, 8, 4, 2):
        if B % cand == 0:
            bn = cand
            break
    return pl.pallas_call(
        _conv_tower_kernel,
        out_shape=jax.ShapeDtypeStruct((B, 25, 64), _BF),
        grid=(B // bn,),
        in_specs=[
            pl.BlockSpec((bn, 512, 16), lambda b: (b, 0, 0)),
            pl.BlockSpec((200, 64), lambda b: (0, 0)),
            pl.BlockSpec((1, 64), lambda b: (0, 0)),
            pl.BlockSpec((1600, 64), lambda b: (0, 0)),
            pl.BlockSpec((1, 64), lambda b: (0, 0)),
        ],
        out_specs=pl.BlockSpec((bn, 25, 64), lambda b: (b, 0, 0)),
        compiler_params=pltpu.CompilerParams(
            dimension_semantics=("parallel",)),
    )(x_bf, w1, b1, w2, b2)


def _fc_stack_kernel(x_ref, w1_ref, b1_ref, w2_ref, b2_ref, w3_ref, b3_ref,
                     o_ref):
    h = jnp.dot(x_ref[...], w1_ref[...], preferred_element_type=_F32)
    h = jnp.maximum(h + b1_ref[...], 0.0).astype(_BF)
    h = jnp.dot(h, w2_ref[...], preferred_element_type=_F32)
    h = jnp.maximum(h + b2_ref[...], 0.0).astype(_BF)
    o_ref[...] = jnp.dot(h, w3_ref[...], preferred_element_type=_F32) \
        + b3_ref[...]


def _fc_stack(feats2d, w1, b1, w2, b2, w3, b3):
    B = feats2d.shape[0]
    tm = B
    for cand in (256, 128, 64):
        if B % cand == 0:
            tm = cand
            break
    return pl.pallas_call(
        _fc_stack_kernel,
        out_shape=jax.ShapeDtypeStruct((B, 128), _F32),
        grid=(B // tm,),
        in_specs=[
            pl.BlockSpec((tm, 1600), lambda i: (i, 0)),
            pl.BlockSpec((1600, 384), lambda i: (0, 0)),
            pl.BlockSpec((1, 384), lambda i: (0, 0)),
            pl.BlockSpec((384, 192), lambda i: (0, 0)),
            pl.BlockSpec((1, 192), lambda i: (0, 0)),
            pl.BlockSpec((192, 128), lambda i: (0, 0)),
            pl.BlockSpec((1, 128), lambda i: (0, 0)),
        ],
        out_specs=pl.BlockSpec((tm, 128), lambda i: (i, 0)),
        compiler_params=pltpu.CompilerParams(
            dimension_semantics=("parallel",)),
    )(feats2d, w1, b1, w2, b2, w3, b3)


@jax.jit
def _forward(x_nchw, w1k, b1r, w2k, b2r, fc1a_w, fc1_b, fc2a_w, fc2_b,
             fc3a_w, fc3_b):
    B = x_nchw.shape[0]
    x = jnp.transpose(x_nchw, (0, 2, 3, 1)).reshape(B, 1024, 3)
    x = jnp.pad(x, ((0, 0), (0, 0), (0, 5))).astype(_BF)
    x = x.reshape(B, 512, 16)
    w1 = w1k.reshape(200, 64).astype(_BF)
    w2 = w2k.reshape(1600, 64).astype(_BF)
    feats = _conv_tower(x, w1, b1r, w2, b2r)
    logits = _fc_stack(feats.reshape(B, 1600),
                       fc1a_w.astype(_BF), fc1_b,
                       fc2a_w.astype(_BF), fc2_b,
                       fc3a_w.astype(_BF), fc3_b)
    return logits[:, :10]


def kernel(x_nchw, w1k, b1r, w2k, b2r, fc1a_w, fc1_b, fc2a_w, fc2_b,
           fc3a_w, fc3_b):
    return _forward(x_nchw, w1k, b1r, w2k, b2r, fc1a_w, fc1_b, fc2a_w,
                    fc2_b, fc3a_w, fc3_b)


# speedup vs baseline: 6.3824x; 1.4205x over previous
import jax
import jax.numpy as jnp
from jax.experimental import pallas as pl
from jax.experimental.pallas import tpu as pltpu

_BF = jnp.bfloat16
_F32 = jnp.float32


def _build_im1(srcs):
    cols = []
    for src, sh in srcs:
        col = jax.lax.slice(src, (sh, 0), (512, 8))
        if sh:
            col = jnp.pad(col, ((0, sh), (0, 0)))
        cols.append(col)
    xts = jnp.concatenate(cols, axis=1)
    return jnp.concatenate(
        [jax.lax.slice(xts, (16 * kh, 0), (16 * kh + 448, 40))
         for kh in range(5)],
        axis=1)


def _one_image(xr, w1, b1, w2, b2):
    xe = xr[:, 0:8]
    xo = xr[:, 8:16]
    ime = _build_im1([(xe, 0), (xo, 0), (xe, 1), (xo, 1), (xe, 2)])
    imo = _build_im1([(xo, 0), (xe, 1), (xo, 1), (xe, 2), (xo, 2)])
    pe = jnp.maximum(
        jnp.dot(ime, w1, preferred_element_type=_F32) + b1, 0.0).astype(_BF)
    po = jnp.maximum(
        jnp.dot(imo, w1, preferred_element_type=_F32) + b1, 0.0).astype(_BF)
    p1 = jnp.maximum(
        jnp.maximum(pe[0:448, :], po[0:448, :]),
        jnp.maximum(jnp.pad(pe[16:448, :], ((0, 16), (0, 0))),
                    jnp.pad(po[16:448, :], ((0, 16), (0, 0)))))
    p1c = jax.lax.slice(p1.reshape(14, 32, 64), (0, 0, 0),
                        (14, 16, 64)).reshape(224, 64)
    im2 = jnp.concatenate(
        [jax.lax.slice(p1c, (16 * kh + kw, 0), (16 * kh + kw + 154, 64))
         for kh in range(5) for kw in range(5)],
        axis=1)
    c2 = jnp.dot(im2, w2, preferred_element_type=_F32)
    c3 = c2.reshape(77, 2, 64)
    se = c3[:, 0, :]
    so = c3[:, 1, :]
    q = jnp.maximum(
        jnp.maximum(se[0:69, :], so[0:69, :]),
        jnp.maximum(se[8:77, :], so[8:77, :]))
    rows = [jax.lax.slice(q, (16 * hp, 0), (16 * hp + 5, 64))
            for hp in range(5)]
    feats = jnp.concatenate(rows, axis=0)
    return jnp.maximum(feats + b2, 0.0).astype(_BF)


def _conv_tower_kernel(x_ref, w1_ref, b1_ref, w2_ref, b2_ref, f_ref):
    bn = x_ref.shape[0]
    w1 = w1_ref[...]
    b1 = b1_ref[...]
    w2 = w2_ref[...]
    b2 = b2_ref[...]

    def body(i, carry):
        f_ref[2 * i] = _one_image(x_ref[2 * i], w1, b1, w2, b2)
        f_ref[2 * i + 1] = _one_image(x_ref[2 * i + 1], w1, b1, w2, b2)
        return carry

    jax.lax.fori_loop(0, bn // 2, body, 0)


def _conv_tower(x_bf, w1, b1, w2, b2):
    B = x_bf.shape[0]
    bn = 1
    for cand in (16, 8, 4, 2):
        if B % cand == 0:
            bn = cand
            break
    return pl.pallas_call(
        _conv_tower_kernel,
        out_shape=jax.ShapeDtypeStruct((B, 25, 64), _BF),
        grid=(B // bn,),
        in_specs=[
            pl.BlockSpec((bn, 512, 16), lambda b: (b, 0, 0)),
            pl.BlockSpec((200, 64), lambda b: (0, 0)),
            pl.BlockSpec((1, 64), lambda b: (0, 0)),
            pl.BlockSpec((1600, 64), lambda b: (0, 0)),
            pl.BlockSpec((1, 64), lambda b: (0, 0)),
        ],
        out_specs=pl.BlockSpec((bn, 25, 64), lambda b: (b, 0, 0)),
        compiler_params=pltpu.CompilerParams(
            dimension_semantics=("parallel",)),
    )(x_bf, w1, b1, w2, b2)


def _fc_stack_kernel(x_ref, w1_ref, b1_ref, w2_ref, b2_ref, w3_ref, b3_ref,
                     o_ref):
    h = jnp.dot(x_ref[...], w1_ref[...], preferred_element_type=_F32)
    h = jnp.maximum(h + b1_ref[...], 0.0).astype(_BF)
    h = jnp.dot(h, w2_ref[...], preferred_element_type=_F32)
    h = jnp.maximum(h + b2_ref[...], 0.0).astype(_BF)
    o_ref[...] = jnp.dot(h, w3_ref[...], preferred_element_type=_F32) \
        + b3_ref[...]


def _fc_stack(feats2d, w1, b1, w2, b2, w3, b3):
    B = feats2d.shape[0]
    tm = B
    for cand in (256, 128, 64):
        if B % cand == 0:
            tm = cand
            break
    return pl.pallas_call(
        _fc_stack_kernel,
        out_shape=jax.ShapeDtypeStruct((B, 128), _F32),
        grid=(B // tm,),
        in_specs=[
            pl.BlockSpec((tm, 1600), lambda i: (i, 0)),
            pl.BlockSpec((1600, 384), lambda i: (0, 0)),
            pl.BlockSpec((1, 384), lambda i: (0, 0)),
            pl.BlockSpec((384, 192), lambda i: (0, 0)),
            pl.BlockSpec((1, 192), lambda i: (0, 0)),
            pl.BlockSpec((192, 128), lambda i: (0, 0)),
            pl.BlockSpec((1, 128), lambda i: (0, 0)),
        ],
        out_specs=pl.BlockSpec((tm, 128), lambda i: (i, 0)),
        compiler_params=pltpu.CompilerParams(
            dimension_semantics=("parallel",)),
    )(feats2d, w1, b1, w2, b2, w3, b3)


@jax.jit
def _forward(x_nchw, w1k, b1r, w2k, b2r, fc1a_w, fc1_b, fc2a_w, fc2_b,
             fc3a_w, fc3_b):
    B = x_nchw.shape[0]
    x = jnp.transpose(x_nchw, (0, 2, 3, 1)).reshape(B, 1024, 3)
    x = jnp.pad(x, ((0, 0), (0, 0), (0, 5))).astype(_BF)
    x = x.reshape(B, 512, 16)
    w1 = w1k.reshape(200, 64).astype(_BF)
    w2 = w2k.reshape(1600, 64).astype(_BF)
    feats = _conv_tower(x, w1, b1r, w2, b2r)
    logits = _fc_stack(feats.reshape(B, 1600),
                       fc1a_w.astype(_BF), fc1_b,
                       fc2a_w.astype(_BF), fc2_b,
                       fc3a_w.astype(_BF), fc3_b)
    return logits[:, :10]


def kernel(x_nchw, w1k, b1r, w2k, b2r, fc1a_w, fc1_b, fc2a_w, fc2_b,
           fc3a_w, fc3_b):
    return _forward(x_nchw, w1k, b1r, w2k, b2r, fc1a_w, fc1_b, fc2a_w,
                    fc2_b, fc3a_w, fc3_b)


# speedup vs baseline: 6.4026x; 1.0032x over previous
import jax
import jax.numpy as jnp
from jax.experimental import pallas as pl
from jax.experimental.pallas import tpu as pltpu

_BF = jnp.bfloat16
_F32 = jnp.float32


def _build_im1(srcs):
    cols = []
    for src, sh in srcs:
        col = jax.lax.slice(src, (sh, 0), (512, 8))
        if sh:
            col = jnp.pad(col, ((0, sh), (0, 0)))
        cols.append(col)
    xts = jnp.concatenate(cols, axis=1)
    return jnp.concatenate(
        [jax.lax.slice(xts, (16 * kh, 0), (16 * kh + 448, 40))
         for kh in range(5)],
        axis=1)


def _one_image(xr, w1, b1, w2, b2):
    xe = xr[:, 0:8]
    xo = xr[:, 8:16]
    ime = _build_im1([(xe, 0), (xo, 0), (xe, 1), (xo, 1), (xe, 2)])
    imo = _build_im1([(xo, 0), (xe, 1), (xo, 1), (xe, 2), (xo, 2)])
    pe = jnp.maximum(
        jnp.dot(ime, w1, preferred_element_type=_F32) + b1, 0.0).astype(_BF)
    po = jnp.maximum(
        jnp.dot(imo, w1, preferred_element_type=_F32) + b1, 0.0).astype(_BF)
    p1 = jnp.maximum(
        jnp.maximum(pe[0:448, :], po[0:448, :]),
        jnp.maximum(jnp.pad(pe[16:448, :], ((0, 16), (0, 0))),
                    jnp.pad(po[16:448, :], ((0, 16), (0, 0)))))
    p1c = jax.lax.slice(p1.reshape(14, 32, 64), (0, 0, 0),
                        (14, 16, 64)).reshape(224, 64)
    im2 = jnp.concatenate(
        [jax.lax.slice(p1c, (16 * kh + kw, 0), (16 * kh + kw + 154, 64))
         for kh in range(5) for kw in range(5)],
        axis=1)
    c2 = jnp.dot(im2, w2, preferred_element_type=_F32)
    c3 = c2.reshape(77, 2, 64)
    se = c3[:, 0, :]
    so = c3[:, 1, :]
    q = jnp.maximum(
        jnp.maximum(se[0:69, :], so[0:69, :]),
        jnp.maximum(se[8:77, :], so[8:77, :]))
    rows = [jax.lax.slice(q, (16 * hp, 0), (16 * hp + 5, 64))
            for hp in range(5)]
    feats = jnp.concatenate(rows, axis=0)
    return jnp.maximum(feats + b2, 0.0).astype(_BF)


def _conv_tower_kernel(x_ref, w1_ref, b1_ref, w2_ref, b2_ref, f_ref):
    bn = x_ref.shape[0]
    w1 = w1_ref[...]
    b1 = b1_ref[...]
    w2 = w2_ref[...]
    b2 = b2_ref[...]

    def body(i, carry):
        for g in range(4):
            f_ref[4 * i + g] = _one_image(x_ref[4 * i + g], w1, b1, w2, b2)
        return carry

    jax.lax.fori_loop(0, bn // 4, body, 0)


def _conv_tower(x_bf, w1, b1, w2, b2):
    B = x_bf.shape[0]
    bn = 1
    for cand in (16, 8, 4, 2):
        if B % cand == 0:
            bn = cand
            break
    return pl.pallas_call(
        _conv_tower_kernel,
        out_shape=jax.ShapeDtypeStruct((B, 25, 64), _BF),
        grid=(B // bn,),
        in_specs=[
            pl.BlockSpec((bn, 512, 16), lambda b: (b, 0, 0)),
            pl.BlockSpec((200, 64), lambda b: (0, 0)),
            pl.BlockSpec((1, 64), lambda b: (0, 0)),
            pl.BlockSpec((1600, 64), lambda b: (0, 0)),
            pl.BlockSpec((1, 64), lambda b: (0, 0)),
        ],
        out_specs=pl.BlockSpec((bn, 25, 64), lambda b: (b, 0, 0)),
        compiler_params=pltpu.CompilerParams(
            dimension_semantics=("parallel",)),
    )(x_bf, w1, b1, w2, b2)


def _fc_stack_kernel(x_ref, w1_ref, b1_ref, w2_ref, b2_ref, w3_ref, b3_ref,
                     o_ref):
    h = jnp.dot(x_ref[...], w1_ref[...], preferred_element_type=_F32)
    h = jnp.maximum(h + b1_ref[...], 0.0).astype(_BF)
    h = jnp.dot(h, w2_ref[...], preferred_element_type=_F32)
    h = jnp.maximum(h + b2_ref[...], 0.0).astype(_BF)
    o_ref[...] = jnp.dot(h, w3_ref[...], preferred_element_type=_F32) \
        + b3_ref[...]


def _fc_stack(feats2d, w1, b1, w2, b2, w3, b3):
    B = feats2d.shape[0]
    tm = B
    for cand in (256, 128, 64):
        if B % cand == 0:
            tm = cand
            break
    return pl.pallas_call(
        _fc_stack_kernel,
        out_shape=jax.ShapeDtypeStruct((B, 128), _F32),
        grid=(B // tm,),
        in_specs=[
            pl.BlockSpec((tm, 1600), lambda i: (i, 0)),
            pl.BlockSpec((1600, 384), lambda i: (0, 0)),
            pl.BlockSpec((1, 384), lambda i: (0, 0)),
            pl.BlockSpec((384, 192), lambda i: (0, 0)),
            pl.BlockSpec((1, 192), lambda i: (0, 0)),
            pl.BlockSpec((192, 128), lambda i: (0, 0)),
            pl.BlockSpec((1, 128), lambda i: (0, 0)),
        ],
        out_specs=pl.BlockSpec((tm, 128), lambda i: (i, 0)),
        compiler_params=pltpu.CompilerParams(
            dimension_semantics=("parallel",)),
    )(feats2d, w1, b1, w2, b2, w3, b3)


@jax.jit
def _forward(x_nchw, w1k, b1r, w2k, b2r, fc1a_w, fc1_b, fc2a_w, fc2_b,
             fc3a_w, fc3_b):
    B = x_nchw.shape[0]
    x = jnp.transpose(x_nchw, (0, 2, 3, 1)).reshape(B, 1024, 3)
    x = jnp.pad(x, ((0, 0), (0, 0), (0, 5))).astype(_BF)
    x = x.reshape(B, 512, 16)
    w1 = w1k.reshape(200, 64).astype(_BF)
    w2 = w2k.reshape(1600, 64).astype(_BF)
    feats = _conv_tower(x, w1, b1r, w2, b2r)
    logits = _fc_stack(feats.reshape(B, 1600),
                       fc1a_w.astype(_BF), fc1_b,
                       fc2a_w.astype(_BF), fc2_b,
                       fc3a_w.astype(_BF), fc3_b)
    return logits[:, :10]


def kernel(x_nchw, w1k, b1r, w2k, b2r, fc1a_w, fc1_b, fc2a_w, fc2_b,
           fc3a_w, fc3_b):
    return _forward(x_nchw, w1k, b1r, w2k, b2r, fc1a_w, fc1_b, fc2a_w,
                    fc2_b, fc3a_w, fc3_b)
